# baseline (speedup 1.0000x reference)
import jax
import jax.numpy as jnp
from jax import lax
from jax.experimental import pallas as pl
from jax.experimental.pallas import tpu as pltpu
from jax.experimental.pallas import tpu_sc as plsc

DIM = 32
TOTAL = 1_638_400
LANES = 128
NC, NS = 2, 16
NW = NC * NS
TOK_PER_W = TOTAL // NW
K = 8
CH = K * LANES
N_CHUNKS = TOK_PER_W // CH
NBUF = 2


def _gather_body(table_hbm, idx_hbm, out_hbm, idx_v, rows_v, gsems, ssems):
    wid = lax.axis_index("s") * NC + lax.axis_index("c")
    base0 = wid * TOK_PER_W

    def fire(b, c):
        base = base0 + c * CH
        pltpu.sync_copy(idx_hbm.at[pl.ds(base, CH)], idx_v.at[b])
        for j in range(K):
            pltpu.async_copy(
                table_hbm.at[idx_v.at[b].at[pl.ds(j * LANES, LANES)]],
                rows_v.at[b].at[pl.ds(j * LANES, LANES)],
                gsems.at[b],
            )

    def drain_gathers(b):
        pltpu.make_async_copy(
            out_hbm.at[pl.ds(0, CH)], rows_v.at[b], gsems.at[b]
        ).wait()

    for b in range(NBUF):
        fire(b, b)

    def outer(g, carry):
        c0 = g * NBUF
        for b in range(NBUF):
            drain_gathers(b)
            base = base0 + (c0 + b) * CH
            pltpu.async_copy(rows_v.at[b], out_hbm.at[pl.ds(base, CH)], ssems.at[b])
        for b in range(NBUF):
            pltpu.make_async_copy(
                rows_v.at[b], out_hbm.at[pl.ds(0, CH)], ssems.at[b]
            ).wait()

            @pl.when(c0 + b + NBUF < N_CHUNKS)
            def _():
                fire(b, c0 + b + NBUF)

        return carry

    lax.fori_loop(0, N_CHUNKS // NBUF, outer, 0)


def kernel(x_data, table):
    idx = x_data.astype(jnp.int32)
    mesh = plsc.VectorSubcoreMesh(core_axis_name="c", subcore_axis_name="s")
    f = pl.kernel(
        _gather_body,
        mesh=mesh,
        out_type=jax.ShapeDtypeStruct((TOTAL, DIM), jnp.float32),
        scratch_types=[
            pltpu.VMEM((NBUF, CH), jnp.int32),
            pltpu.VMEM((NBUF, CH, DIM), jnp.float32),
            pltpu.SemaphoreType.DMA((NBUF,)),
            pltpu.SemaphoreType.DMA((NBUF,)),
        ],
        compiler_params=pltpu.CompilerParams(use_tc_tiling_on_sc=False),
    )
    return f(table, idx)
